# baseline (speedup 1.0000x reference)
import functools

import jax
import jax.numpy as jnp
from jax import lax
from jax.experimental import pallas as pl
from jax.experimental.pallas import tpu as pltpu
from jax.experimental.pallas import tpu_sc as plsc

N = 10000
E = 320000
D = 128
DE = 16
MD = 64
ND = 64

NC = 2
NS = 16
NW = NC * NS
EPW = E // NW
CHUNK = 80
NCHUNK = EPW // CHUNK
NPAD = 10240
RPS = NPAD // NS

_SC_MESH = plsc.VectorSubcoreMesh(core_axis_name="c", subcore_axis_name="s")


def _edge_body(hw, ew, src3, dst3, zeros, out,
               src_all, dst_all, e0, e1, r0, r1, acc,
               se0, se1, sg0, sg1, ss0, ss1):
    rows = [r0, r1]
    ews = [e0, e1]
    se = [se0, se1]
    sg = [sg0, sg1]
    ss = [ss0, ss1]

    cid = lax.axis_index("c")
    sid = lax.axis_index("s")
    wid = sid * NC + cid
    base = wid * EPW

    pltpu.sync_copy(zeros.at[pl.ds(sid * RPS, RPS)], acc.at[pl.ds(sid * RPS, RPS)])
    pltpu.sync_copy(src3.at[wid], src_all)
    pltpu.sync_copy(dst3.at[wid], dst_all)
    plsc.subcore_barrier()

    def ew_start(k, bi):
        pltpu.async_copy(ew.at[pl.ds(base + k * CHUNK, CHUNK)], ews[bi], se[bi])

    def ew_wait(k, bi):
        pltpu.make_async_copy(ew.at[pl.ds(base + k * CHUNK, CHUNK)],
                              ews[bi], se[bi]).wait()

    def g_start(k, bi):
        pltpu.async_copy(hw.at[src_all.at[k]], rows[bi], sg[bi])

    def g_wait(k, bi):
        pltpu.make_async_copy(hw.at[src_all.at[k]], rows[bi], sg[bi]).wait()

    def s_start(k, bi):
        pltpu.async_copy(rows[bi], acc.at[dst_all.at[k]], ss[bi], add=True)

    def s_wait(k, bi):
        pltpu.make_async_copy(rows[bi], acc.at[dst_all.at[k]], ss[bi]).wait()

    def compute(bi):
        @plsc.parallel_loop(0, CHUNK, 1, unroll=2)
        def _(j):
            for q in range(MD // 16):
                v = rows[bi][j, pl.ds(q * 16, 16)] + ews[bi][j, pl.ds(q * 16, 16)]
                t = jnp.exp(v)
                rows[bi][j, pl.ds(q * 16, 16)] = 1.0 - 2.0 / (t + 1.0)

    ew_start(0, 0)
    g_start(0, 0)

    def pair_body(p, carry):
        for b in range(2):
            k = p * 2 + b
            ew_wait(k, b)
            g_wait(k, b)
            compute(b)
            s_start(k, b)

            @pl.when(k >= 1)
            def _():
                s_wait(k - 1, 1 - b)

            ew_start(k + 1, 1 - b)
            g_start(k + 1, 1 - b)
        return carry

    lax.fori_loop(0, (NCHUNK - 1) // 2, pair_body, 0)
    kt = NCHUNK - 1
    ew_wait(kt, 0)
    g_wait(kt, 0)
    compute(0)
    s_start(kt, 0)
    s_wait(kt - 1, 1)
    s_wait(kt, 0)

    plsc.subcore_barrier()
    pltpu.sync_copy(acc.at[pl.ds(sid * RPS, RPS)],
                    out.at[cid, pl.ds(sid * RPS, RPS)])


_edge_pass = pl.kernel(
    _edge_body,
    out_type=jax.ShapeDtypeStruct((NC, NPAD, MD), jnp.float32),
    mesh=_SC_MESH,
    scratch_types=[
        pltpu.VMEM((NCHUNK, CHUNK), jnp.int32),
        pltpu.VMEM((NCHUNK, CHUNK), jnp.int32),
        pltpu.VMEM((CHUNK, MD), jnp.float32),
        pltpu.VMEM((CHUNK, MD), jnp.float32),
        pltpu.VMEM((CHUNK, MD), jnp.float32),
        pltpu.VMEM((CHUNK, MD), jnp.float32),
        pltpu.VMEM_SHARED((NPAD, MD), jnp.float32),
        pltpu.SemaphoreType.DMA,
        pltpu.SemaphoreType.DMA,
        pltpu.SemaphoreType.DMA,
        pltpu.SemaphoreType.DMA,
        pltpu.SemaphoreType.DMA,
        pltpu.SemaphoreType.DMA,
    ],
    compiler_params=pltpu.CompilerParams(use_tc_tiling_on_sc=False),
)


_BE = 8000
_BN = 2000


def _pre_body(ea_ref, w1b_ref, x_ref, w1a_ref, b1_ref, ew1_ref, xw1_ref):
    ew1_ref[...] = jnp.dot(ea_ref[...], w1b_ref[...],
                           preferred_element_type=jnp.float32)
    xw1_ref[...] = jnp.dot(x_ref[...], w1a_ref[...],
                           preferred_element_type=jnp.float32) + b1_ref[...]


def _pre_call(edge_attr, w1b, x, w1a, b1):
    return pl.pallas_call(
        _pre_body,
        grid=(E // _BE,),
        in_specs=[
            pl.BlockSpec((_BE, DE), lambda i: (i, 0)),
            pl.BlockSpec((DE, MD), lambda i: (0, 0)),
            pl.BlockSpec((_BN, D), lambda i: (jnp.minimum(i, 4), 0)),
            pl.BlockSpec((D, MD), lambda i: (0, 0)),
            pl.BlockSpec((1, MD), lambda i: (0, 0)),
        ],
        out_specs=[
            pl.BlockSpec((_BE, MD), lambda i: (i, 0)),
            pl.BlockSpec((_BN, MD), lambda i: (jnp.minimum(i, 4), 0)),
        ],
        out_shape=[
            jax.ShapeDtypeStruct((E, MD), jnp.float32),
            jax.ShapeDtypeStruct((N, MD), jnp.float32),
        ],
    )(edge_attr, w1b, x, w1a, b1)


def _ew2_body(ea_ref, w2b_ref, ew2_ref):
    ew2_ref[...] = jnp.dot(ea_ref[...], w2b_ref[...],
                           preferred_element_type=jnp.float32)


def _ew2_call(edge_attr, w2b):
    return pl.pallas_call(
        _ew2_body,
        grid=(E // _BE,),
        in_specs=[
            pl.BlockSpec((_BE, DE), lambda i: (i, 0)),
            pl.BlockSpec((DE, MD), lambda i: (0, 0)),
        ],
        out_specs=pl.BlockSpec((_BE, MD), lambda i: (i, 0)),
        out_shape=jax.ShapeDtypeStruct((E, MD), jnp.float32),
    )(edge_attr, w2b)


def _h1_body(x_ref, pp_ref, wna_ref, wnb_ref, bn_ref, w2a_ref, b2_ref,
             h_ref, xw2_ref):
    pooled = pp_ref[0] + pp_ref[1]
    h = jnp.tanh(jnp.dot(x_ref[...], wna_ref[...], preferred_element_type=jnp.float32)
                 + jnp.dot(pooled, wnb_ref[...], preferred_element_type=jnp.float32)
                 + bn_ref[...])
    h_ref[...] = h
    xw2_ref[...] = jnp.dot(h, w2a_ref[...],
                           preferred_element_type=jnp.float32) + b2_ref[...]


def _h1_call(x, pp, wn1a, wn1b, bn1, w2a, b2):
    return pl.pallas_call(
        _h1_body,
        grid=(N // _BN,),
        in_specs=[
            pl.BlockSpec((_BN, D), lambda i: (i, 0)),
            pl.BlockSpec((NC, _BN, MD), lambda i: (0, i, 0)),
            pl.BlockSpec((D, ND), lambda i: (0, 0)),
            pl.BlockSpec((MD, ND), lambda i: (0, 0)),
            pl.BlockSpec((1, ND), lambda i: (0, 0)),
            pl.BlockSpec((ND, MD), lambda i: (0, 0)),
            pl.BlockSpec((1, MD), lambda i: (0, 0)),
        ],
        out_specs=[
            pl.BlockSpec((_BN, ND), lambda i: (i, 0)),
            pl.BlockSpec((_BN, MD), lambda i: (i, 0)),
        ],
        out_shape=[
            jax.ShapeDtypeStruct((N, ND), jnp.float32),
            jax.ShapeDtypeStruct((N, MD), jnp.float32),
        ],
    )(x, pp, wn1a, wn1b, bn1, w2a, b2)


def _out_body(h_ref, pp_ref, wna_ref, wnb_ref, bn_ref, wo_ref, bo_ref, o_ref):
    pooled = pp_ref[0] + pp_ref[1]
    h2 = jnp.tanh(jnp.dot(h_ref[...], wna_ref[...], preferred_element_type=jnp.float32)
                  + jnp.dot(pooled, wnb_ref[...], preferred_element_type=jnp.float32)
                  + bn_ref[...])
    o_ref[...] = jnp.dot(h2, wo_ref[...],
                         preferred_element_type=jnp.float32) + bo_ref[...]


def _out_call(h, pp, wn2a, wn2b, bn2, wo, bo):
    return pl.pallas_call(
        _out_body,
        grid=(N // _BN,),
        in_specs=[
            pl.BlockSpec((_BN, ND), lambda i: (i, 0)),
            pl.BlockSpec((NC, _BN, MD), lambda i: (0, i, 0)),
            pl.BlockSpec((ND, ND), lambda i: (0, 0)),
            pl.BlockSpec((MD, ND), lambda i: (0, 0)),
            pl.BlockSpec((1, ND), lambda i: (0, 0)),
            pl.BlockSpec((ND, 1), lambda i: (0, 0)),
            pl.BlockSpec((1, 1), lambda i: (0, 0)),
        ],
        out_specs=pl.BlockSpec((_BN, 1), lambda i: (i, 0)),
        out_shape=jax.ShapeDtypeStruct((N, 1), jnp.float32),
    )(h, pp, wn2a, wn2b, bn2, wo, bo)


def kernel(x, edge_index, edge_attr, W1, b1, Wn1, bn1, W2, b2, Wn2, bn2, Wo, bo):
    src3 = edge_index[0].reshape(NW, NCHUNK, CHUNK)
    dst3 = edge_index[1].reshape(NW, NCHUNK, CHUNK)
    zeros = jnp.zeros((NPAD, MD), jnp.float32)

    ew1, xw1 = _pre_call(edge_attr, 2.0 * W1[D:], x, 2.0 * W1[:D],
                         2.0 * b1.reshape(1, MD))
    pp1 = _edge_pass(xw1, ew1, src3, dst3, zeros)
    ew2 = _ew2_call(edge_attr, 2.0 * W2[ND:])
    h, xw2 = _h1_call(x, pp1, Wn1[:D], Wn1[D:], bn1.reshape(1, ND),
                      2.0 * W2[:ND], 2.0 * b2.reshape(1, MD))
    pp2 = _edge_pass(xw2, ew2, src3, dst3, zeros)
    return _out_call(h, pp2, Wn2[:ND], Wn2[ND:], bn2.reshape(1, ND),
                     Wo, bo.reshape(1, 1))


# speedup vs baseline: 1.0111x; 1.0111x over previous
import functools

import jax
import jax.numpy as jnp
from jax import lax
from jax.experimental import pallas as pl
from jax.experimental.pallas import tpu as pltpu
from jax.experimental.pallas import tpu_sc as plsc

N = 10000
E = 320000
D = 128
DE = 16
MD = 64
ND = 64

NC = 2
NS = 16
NW = NC * NS
EPW = E // NW
CHUNK = 80
NCHUNK = EPW // CHUNK
NPAD = 10240
RPS = NPAD // NS

_SC_MESH = plsc.VectorSubcoreMesh(core_axis_name="c", subcore_axis_name="s")


def _edge_body(hw, ew, src3, dst3, zeros, out,
               src_all, dst_all, e0, e1, r0, r1, acc,
               se0, se1, sg0, sg1, ss0, ss1):
    rows = [r0, r1]
    ews = [e0, e1]
    se = [se0, se1]
    sg = [sg0, sg1]
    ss = [ss0, ss1]

    cid = lax.axis_index("c")
    sid = lax.axis_index("s")
    wid = sid * NC + cid
    base = wid * EPW

    pltpu.sync_copy(zeros.at[pl.ds(sid * RPS, RPS)], acc.at[pl.ds(sid * RPS, RPS)])
    pltpu.sync_copy(src3.at[wid], src_all)
    pltpu.sync_copy(dst3.at[wid], dst_all)
    plsc.subcore_barrier()

    def ew_start(k, bi):
        pltpu.async_copy(ew.at[pl.ds(base + k * CHUNK, CHUNK)], ews[bi], se[bi])

    def ew_wait(k, bi):
        pltpu.make_async_copy(ew.at[pl.ds(base + k * CHUNK, CHUNK)],
                              ews[bi], se[bi]).wait()

    def g_start(k, bi):
        pltpu.async_copy(hw.at[src_all.at[k]], rows[bi], sg[bi])

    def g_wait(k, bi):
        pltpu.make_async_copy(hw.at[src_all.at[k]], rows[bi], sg[bi]).wait()

    def s_start(k, bi):
        pltpu.async_copy(rows[bi], acc.at[dst_all.at[k]], ss[bi], add=True)

    def s_wait(k, bi):
        pltpu.make_async_copy(rows[bi], acc.at[dst_all.at[k]], ss[bi]).wait()

    def compute(bi):
        @plsc.parallel_loop(0, CHUNK, 1)
        def _(j):
            for q in range(MD // 16):
                v = rows[bi][j, pl.ds(q * 16, 16)] + ews[bi][j, pl.ds(q * 16, 16)]
                t = jnp.exp(v)
                rows[bi][j, pl.ds(q * 16, 16)] = 1.0 - 2.0 / (t + 1.0)

    ew_start(0, 0)
    g_start(0, 0)

    def pair_body(p, carry):
        for b in range(2):
            k = p * 2 + b
            ew_wait(k, b)
            g_wait(k, b)
            compute(b)
            s_start(k, b)

            @pl.when(k >= 1)
            def _():
                s_wait(k - 1, 1 - b)

            ew_start(k + 1, 1 - b)
            g_start(k + 1, 1 - b)
        return carry

    lax.fori_loop(0, (NCHUNK - 1) // 2, pair_body, 0)
    kt = NCHUNK - 1
    ew_wait(kt, 0)
    g_wait(kt, 0)
    compute(0)
    s_start(kt, 0)
    s_wait(kt - 1, 1)
    s_wait(kt, 0)

    plsc.subcore_barrier()
    pltpu.sync_copy(acc.at[pl.ds(sid * RPS, RPS)],
                    out.at[cid, pl.ds(sid * RPS, RPS)])


_edge_pass = pl.kernel(
    _edge_body,
    out_type=jax.ShapeDtypeStruct((NC, NPAD, MD), jnp.float32),
    mesh=_SC_MESH,
    scratch_types=[
        pltpu.VMEM((NCHUNK, CHUNK), jnp.int32),
        pltpu.VMEM((NCHUNK, CHUNK), jnp.int32),
        pltpu.VMEM((CHUNK, MD), jnp.float32),
        pltpu.VMEM((CHUNK, MD), jnp.float32),
        pltpu.VMEM((CHUNK, MD), jnp.float32),
        pltpu.VMEM((CHUNK, MD), jnp.float32),
        pltpu.VMEM_SHARED((NPAD, MD), jnp.float32),
        pltpu.SemaphoreType.DMA,
        pltpu.SemaphoreType.DMA,
        pltpu.SemaphoreType.DMA,
        pltpu.SemaphoreType.DMA,
        pltpu.SemaphoreType.DMA,
        pltpu.SemaphoreType.DMA,
    ],
    compiler_params=pltpu.CompilerParams(use_tc_tiling_on_sc=False),
)


_BE = 8000
_BN = 2000


def _pre_body(ea_ref, w1b_ref, x_ref, w1a_ref, b1_ref, ew1_ref, xw1_ref):
    ew1_ref[...] = jnp.dot(ea_ref[...], w1b_ref[...],
                           preferred_element_type=jnp.float32)
    xw1_ref[...] = jnp.dot(x_ref[...], w1a_ref[...],
                           preferred_element_type=jnp.float32) + b1_ref[...]


def _pre_call(edge_attr, w1b, x, w1a, b1):
    return pl.pallas_call(
        _pre_body,
        grid=(E // _BE,),
        in_specs=[
            pl.BlockSpec((_BE, DE), lambda i: (i, 0)),
            pl.BlockSpec((DE, MD), lambda i: (0, 0)),
            pl.BlockSpec((_BN, D), lambda i: (jnp.minimum(i, 4), 0)),
            pl.BlockSpec((D, MD), lambda i: (0, 0)),
            pl.BlockSpec((1, MD), lambda i: (0, 0)),
        ],
        out_specs=[
            pl.BlockSpec((_BE, MD), lambda i: (i, 0)),
            pl.BlockSpec((_BN, MD), lambda i: (jnp.minimum(i, 4), 0)),
        ],
        out_shape=[
            jax.ShapeDtypeStruct((E, MD), jnp.float32),
            jax.ShapeDtypeStruct((N, MD), jnp.float32),
        ],
    )(edge_attr, w1b, x, w1a, b1)


def _ew2_body(ea_ref, w2b_ref, ew2_ref):
    ew2_ref[...] = jnp.dot(ea_ref[...], w2b_ref[...],
                           preferred_element_type=jnp.float32)


def _ew2_call(edge_attr, w2b):
    return pl.pallas_call(
        _ew2_body,
        grid=(E // _BE,),
        in_specs=[
            pl.BlockSpec((_BE, DE), lambda i: (i, 0)),
            pl.BlockSpec((DE, MD), lambda i: (0, 0)),
        ],
        out_specs=pl.BlockSpec((_BE, MD), lambda i: (i, 0)),
        out_shape=jax.ShapeDtypeStruct((E, MD), jnp.float32),
    )(edge_attr, w2b)


def _h1_body(x_ref, pp_ref, wna_ref, wnb_ref, bn_ref, w2a_ref, b2_ref,
             h_ref, xw2_ref):
    pooled = pp_ref[0] + pp_ref[1]
    h = jnp.tanh(jnp.dot(x_ref[...], wna_ref[...], preferred_element_type=jnp.float32)
                 + jnp.dot(pooled, wnb_ref[...], preferred_element_type=jnp.float32)
                 + bn_ref[...])
    h_ref[...] = h
    xw2_ref[...] = jnp.dot(h, w2a_ref[...],
                           preferred_element_type=jnp.float32) + b2_ref[...]


def _h1_call(x, pp, wn1a, wn1b, bn1, w2a, b2):
    return pl.pallas_call(
        _h1_body,
        grid=(N // _BN,),
        in_specs=[
            pl.BlockSpec((_BN, D), lambda i: (i, 0)),
            pl.BlockSpec((NC, _BN, MD), lambda i: (0, i, 0)),
            pl.BlockSpec((D, ND), lambda i: (0, 0)),
            pl.BlockSpec((MD, ND), lambda i: (0, 0)),
            pl.BlockSpec((1, ND), lambda i: (0, 0)),
            pl.BlockSpec((ND, MD), lambda i: (0, 0)),
            pl.BlockSpec((1, MD), lambda i: (0, 0)),
        ],
        out_specs=[
            pl.BlockSpec((_BN, ND), lambda i: (i, 0)),
            pl.BlockSpec((_BN, MD), lambda i: (i, 0)),
        ],
        out_shape=[
            jax.ShapeDtypeStruct((N, ND), jnp.float32),
            jax.ShapeDtypeStruct((N, MD), jnp.float32),
        ],
    )(x, pp, wn1a, wn1b, bn1, w2a, b2)


def _out_body(h_ref, pp_ref, wna_ref, wnb_ref, bn_ref, wo_ref, bo_ref, o_ref):
    pooled = pp_ref[0] + pp_ref[1]
    h2 = jnp.tanh(jnp.dot(h_ref[...], wna_ref[...], preferred_element_type=jnp.float32)
                  + jnp.dot(pooled, wnb_ref[...], preferred_element_type=jnp.float32)
                  + bn_ref[...])
    o_ref[...] = jnp.dot(h2, wo_ref[...],
                         preferred_element_type=jnp.float32) + bo_ref[...]


def _out_call(h, pp, wn2a, wn2b, bn2, wo, bo):
    return pl.pallas_call(
        _out_body,
        grid=(N // _BN,),
        in_specs=[
            pl.BlockSpec((_BN, ND), lambda i: (i, 0)),
            pl.BlockSpec((NC, _BN, MD), lambda i: (0, i, 0)),
            pl.BlockSpec((ND, ND), lambda i: (0, 0)),
            pl.BlockSpec((MD, ND), lambda i: (0, 0)),
            pl.BlockSpec((1, ND), lambda i: (0, 0)),
            pl.BlockSpec((ND, 1), lambda i: (0, 0)),
            pl.BlockSpec((1, 1), lambda i: (0, 0)),
        ],
        out_specs=pl.BlockSpec((_BN, 1), lambda i: (i, 0)),
        out_shape=jax.ShapeDtypeStruct((N, 1), jnp.float32),
    )(h, pp, wn2a, wn2b, bn2, wo, bo)


def kernel(x, edge_index, edge_attr, W1, b1, Wn1, bn1, W2, b2, Wn2, bn2, Wo, bo):
    src3 = edge_index[0].reshape(NW, NCHUNK, CHUNK)
    dst3 = edge_index[1].reshape(NW, NCHUNK, CHUNK)
    zeros = jnp.zeros((NPAD, MD), jnp.float32)

    ew1, xw1 = _pre_call(edge_attr, 2.0 * W1[D:], x, 2.0 * W1[:D],
                         2.0 * b1.reshape(1, MD))
    pp1 = _edge_pass(xw1, ew1, src3, dst3, zeros)
    ew2 = _ew2_call(edge_attr, 2.0 * W2[ND:])
    h, xw2 = _h1_call(x, pp1, Wn1[:D], Wn1[D:], bn1.reshape(1, ND),
                      2.0 * W2[:ND], 2.0 * b2.reshape(1, MD))
    pp2 = _edge_pass(xw2, ew2, src3, dst3, zeros)
    return _out_call(h, pp2, Wn2[:ND], Wn2[ND:], bn2.reshape(1, ND),
                     Wo, bo.reshape(1, 1))


# speedup vs baseline: 1.0128x; 1.0017x over previous
import functools

import jax
import jax.numpy as jnp
from jax import lax
from jax.experimental import pallas as pl
from jax.experimental.pallas import tpu as pltpu
from jax.experimental.pallas import tpu_sc as plsc

N = 10000
E = 320000
D = 128
DE = 16
MD = 64
ND = 64

NC = 2
NS = 16
NW = NC * NS
EPW = E // NW
CHUNK = 80
NCHUNK = EPW // CHUNK
NPAD = 10240
RPS = NPAD // NS

_SC_MESH = plsc.VectorSubcoreMesh(core_axis_name="c", subcore_axis_name="s")


def _edge_body(hw, ew, src3, dst3, zeros, out,
               src_all, dst_all, e0, e1, r0, r1, acc,
               se0, se1, sg0, sg1, ss0, ss1):
    rows = [r0, r1]
    ews = [e0, e1]
    se = [se0, se1]
    sg = [sg0, sg1]
    ss = [ss0, ss1]

    cid = lax.axis_index("c")
    sid = lax.axis_index("s")
    wid = sid * NC + cid
    base = wid * EPW

    pltpu.sync_copy(zeros.at[pl.ds(sid * RPS, RPS)], acc.at[pl.ds(sid * RPS, RPS)])
    pltpu.sync_copy(src3.at[wid], src_all)
    pltpu.sync_copy(dst3.at[wid], dst_all)
    plsc.subcore_barrier()

    def ew_start(k, bi):
        pltpu.async_copy(ew.at[pl.ds(base + k * CHUNK, CHUNK)], ews[bi], se[bi])

    def ew_wait(k, bi):
        pltpu.make_async_copy(ew.at[pl.ds(base + k * CHUNK, CHUNK)],
                              ews[bi], se[bi]).wait()

    def g_start(k, bi):
        pltpu.async_copy(hw.at[src_all.at[k]], rows[bi], sg[bi])

    def g_wait(k, bi):
        pltpu.make_async_copy(hw.at[src_all.at[k]], rows[bi], sg[bi]).wait()

    def s_start(k, bi):
        pltpu.async_copy(rows[bi], acc.at[dst_all.at[k]], ss[bi], add=True)

    def s_wait(k, bi):
        pltpu.make_async_copy(rows[bi], acc.at[dst_all.at[k]], ss[bi]).wait()

    def compute(bi):
        @plsc.parallel_loop(0, CHUNK, 1)
        def _(j):
            for q in range(MD // 16):
                v = rows[bi][j, pl.ds(q * 16, 16)] + ews[bi][j, pl.ds(q * 16, 16)]
                t = jnp.exp(v)
                rows[bi][j, pl.ds(q * 16, 16)] = 1.0 - 2.0 / (t + 1.0)

    ew_start(0, 0)
    g_start(0, 0)

    def pair_body(p, carry):
        for b in range(2):
            k = p * 2 + b
            ew_wait(k, b)
            g_wait(k, b)
            compute(b)
            s_start(k, b)

            @pl.when(k >= 1)
            def _():
                s_wait(k - 1, 1 - b)

            ew_start(k + 1, 1 - b)
            g_start(k + 1, 1 - b)
        return carry

    lax.fori_loop(0, (NCHUNK - 1) // 2, pair_body, 0)
    kt = NCHUNK - 1
    ew_wait(kt, 0)
    g_wait(kt, 0)
    compute(0)
    s_start(kt, 0)
    s_wait(kt - 1, 1)
    s_wait(kt, 0)

    plsc.subcore_barrier()
    pltpu.sync_copy(acc.at[pl.ds(sid * RPS, RPS)],
                    out.at[cid, pl.ds(sid * RPS, RPS)])


_edge_pass = pl.kernel(
    _edge_body,
    out_type=jax.ShapeDtypeStruct((NC, NPAD, MD), jnp.float32),
    mesh=_SC_MESH,
    scratch_types=[
        pltpu.VMEM((NCHUNK, CHUNK), jnp.int32),
        pltpu.VMEM((NCHUNK, CHUNK), jnp.int32),
        pltpu.VMEM((CHUNK, MD), jnp.float32),
        pltpu.VMEM((CHUNK, MD), jnp.float32),
        pltpu.VMEM((CHUNK, MD), jnp.float32),
        pltpu.VMEM((CHUNK, MD), jnp.float32),
        pltpu.VMEM_SHARED((NPAD, MD), jnp.float32),
        pltpu.SemaphoreType.DMA,
        pltpu.SemaphoreType.DMA,
        pltpu.SemaphoreType.DMA,
        pltpu.SemaphoreType.DMA,
        pltpu.SemaphoreType.DMA,
        pltpu.SemaphoreType.DMA,
    ],
    compiler_params=pltpu.CompilerParams(use_tc_tiling_on_sc=False),
)


_BE = 8000
_BN = 2000


def _pre_body(ea_ref, w1b_ref, x_ref, w1a_ref, b1_ref, w2b_ref,
              ew1_ref, xw1_ref, ew2_ref):
    ew1_ref[...] = jnp.dot(ea_ref[...], w1b_ref[...],
                           preferred_element_type=jnp.float32)
    xw1_ref[...] = jnp.dot(x_ref[...], w1a_ref[...],
                           preferred_element_type=jnp.float32) + b1_ref[...]
    ew2_ref[...] = jnp.dot(ea_ref[...], w2b_ref[...],
                           preferred_element_type=jnp.float32)


def _pre_call(edge_attr, w1b, x, w1a, b1, w2b):
    return pl.pallas_call(
        _pre_body,
        grid=(E // _BE,),
        in_specs=[
            pl.BlockSpec((_BE, DE), lambda i: (i, 0)),
            pl.BlockSpec((DE, MD), lambda i: (0, 0)),
            pl.BlockSpec((_BN, D), lambda i: (jnp.minimum(i, 4), 0)),
            pl.BlockSpec((D, MD), lambda i: (0, 0)),
            pl.BlockSpec((1, MD), lambda i: (0, 0)),
            pl.BlockSpec((DE, MD), lambda i: (0, 0)),
        ],
        out_specs=[
            pl.BlockSpec((_BE, MD), lambda i: (i, 0)),
            pl.BlockSpec((_BN, MD), lambda i: (jnp.minimum(i, 4), 0)),
            pl.BlockSpec((_BE, MD), lambda i: (i, 0)),
        ],
        out_shape=[
            jax.ShapeDtypeStruct((E, MD), jnp.float32),
            jax.ShapeDtypeStruct((N, MD), jnp.float32),
            jax.ShapeDtypeStruct((E, MD), jnp.float32),
        ],
    )(edge_attr, w1b, x, w1a, b1, w2b)


def _ew2_body(ea_ref, w2b_ref, ew2_ref):
    ew2_ref[...] = jnp.dot(ea_ref[...], w2b_ref[...],
                           preferred_element_type=jnp.float32)


def _ew2_call(edge_attr, w2b):
    return pl.pallas_call(
        _ew2_body,
        grid=(E // _BE,),
        in_specs=[
            pl.BlockSpec((_BE, DE), lambda i: (i, 0)),
            pl.BlockSpec((DE, MD), lambda i: (0, 0)),
        ],
        out_specs=pl.BlockSpec((_BE, MD), lambda i: (i, 0)),
        out_shape=jax.ShapeDtypeStruct((E, MD), jnp.float32),
    )(edge_attr, w2b)


def _h1_body(x_ref, pp_ref, wna_ref, wnb_ref, bn_ref, w2a_ref, b2_ref,
             h_ref, xw2_ref):
    pooled = pp_ref[0] + pp_ref[1]
    h = jnp.tanh(jnp.dot(x_ref[...], wna_ref[...], preferred_element_type=jnp.float32)
                 + jnp.dot(pooled, wnb_ref[...], preferred_element_type=jnp.float32)
                 + bn_ref[...])
    h_ref[...] = h
    xw2_ref[...] = jnp.dot(h, w2a_ref[...],
                           preferred_element_type=jnp.float32) + b2_ref[...]


def _h1_call(x, pp, wn1a, wn1b, bn1, w2a, b2):
    return pl.pallas_call(
        _h1_body,
        grid=(N // _BN,),
        in_specs=[
            pl.BlockSpec((_BN, D), lambda i: (i, 0)),
            pl.BlockSpec((NC, _BN, MD), lambda i: (0, i, 0)),
            pl.BlockSpec((D, ND), lambda i: (0, 0)),
            pl.BlockSpec((MD, ND), lambda i: (0, 0)),
            pl.BlockSpec((1, ND), lambda i: (0, 0)),
            pl.BlockSpec((ND, MD), lambda i: (0, 0)),
            pl.BlockSpec((1, MD), lambda i: (0, 0)),
        ],
        out_specs=[
            pl.BlockSpec((_BN, ND), lambda i: (i, 0)),
            pl.BlockSpec((_BN, MD), lambda i: (i, 0)),
        ],
        out_shape=[
            jax.ShapeDtypeStruct((N, ND), jnp.float32),
            jax.ShapeDtypeStruct((N, MD), jnp.float32),
        ],
    )(x, pp, wn1a, wn1b, bn1, w2a, b2)


def _out_body(h_ref, pp_ref, wna_ref, wnb_ref, bn_ref, wo_ref, bo_ref, o_ref):
    pooled = pp_ref[0] + pp_ref[1]
    h2 = jnp.tanh(jnp.dot(h_ref[...], wna_ref[...], preferred_element_type=jnp.float32)
                  + jnp.dot(pooled, wnb_ref[...], preferred_element_type=jnp.float32)
                  + bn_ref[...])
    o_ref[...] = jnp.dot(h2, wo_ref[...],
                         preferred_element_type=jnp.float32) + bo_ref[...]


def _out_call(h, pp, wn2a, wn2b, bn2, wo, bo):
    return pl.pallas_call(
        _out_body,
        grid=(N // _BN,),
        in_specs=[
            pl.BlockSpec((_BN, ND), lambda i: (i, 0)),
            pl.BlockSpec((NC, _BN, MD), lambda i: (0, i, 0)),
            pl.BlockSpec((ND, ND), lambda i: (0, 0)),
            pl.BlockSpec((MD, ND), lambda i: (0, 0)),
            pl.BlockSpec((1, ND), lambda i: (0, 0)),
            pl.BlockSpec((ND, 1), lambda i: (0, 0)),
            pl.BlockSpec((1, 1), lambda i: (0, 0)),
        ],
        out_specs=pl.BlockSpec((_BN, 1), lambda i: (i, 0)),
        out_shape=jax.ShapeDtypeStruct((N, 1), jnp.float32),
    )(h, pp, wn2a, wn2b, bn2, wo, bo)


def kernel(x, edge_index, edge_attr, W1, b1, Wn1, bn1, W2, b2, Wn2, bn2, Wo, bo):
    src3 = edge_index[0].reshape(NW, NCHUNK, CHUNK)
    dst3 = edge_index[1].reshape(NW, NCHUNK, CHUNK)
    zeros = jnp.zeros((NPAD, MD), jnp.float32)

    ew1, xw1, ew2 = _pre_call(edge_attr, 2.0 * W1[D:], x, 2.0 * W1[:D],
                              2.0 * b1.reshape(1, MD), 2.0 * W2[ND:])
    pp1 = _edge_pass(xw1, ew1, src3, dst3, zeros)
    h, xw2 = _h1_call(x, pp1, Wn1[:D], Wn1[D:], bn1.reshape(1, ND),
                      2.0 * W2[:ND], 2.0 * b2.reshape(1, MD))
    pp2 = _edge_pass(xw2, ew2, src3, dst3, zeros)
    return _out_call(h, pp2, Wn2[:ND], Wn2[ND:], bn2.reshape(1, ND),
                     Wo, bo.reshape(1, 1))
